# Initial kernel scaffold; baseline (speedup 1.0000x reference)
import jax
import jax.numpy as jnp
from jax.experimental import pallas as pl


def kernel(user, pos_item, neg_item, user_embedding, item_embedding):
    raise NotImplementedError("write your pallas kernel here")


# speedup vs baseline: 1.6966x; 1.6966x over previous
import functools

import jax
import jax.numpy as jnp
from jax import lax
from jax.experimental import pallas as pl
from jax.experimental.pallas import tpu as pltpu
from jax.experimental.pallas import tpu_sc as plsc

B = 16384
D = 128
NC = 2
NS = 16
L = 16
NW = NC * NS
BPW = B // NW
CH = 128
NCH = BPW // CH

_MESH = plsc.VectorSubcoreMesh(core_axis_name="c", subcore_axis_name="s")


def _bpr_body(user_h, pos_h, neg_h, ue_h, ie_h, pos_o, neg_o,
              idx_v, u0, p0, n0, u1, p1, n1, pout, nout, sem0, sem1):
    wid = lax.axis_index("s") * NC + lax.axis_index("c")
    base = wid * BPW
    ubufs = (u0, u1)
    pbufs = (p0, p1)
    nbufs = (n0, n1)
    sems = (sem0, sem1)

    def fetch(j, slot):
        off = base + j * CH
        pltpu.sync_copy(user_h.at[pl.ds(off, CH)], idx_v.at[slot * 3 + 0])
        pltpu.sync_copy(pos_h.at[pl.ds(off, CH)], idx_v.at[slot * 3 + 1])
        pltpu.sync_copy(neg_h.at[pl.ds(off, CH)], idx_v.at[slot * 3 + 2])
        c1 = pltpu.async_copy(ue_h.at[idx_v.at[slot * 3 + 0]], ubufs[slot], sems[slot])
        c2 = pltpu.async_copy(ie_h.at[idx_v.at[slot * 3 + 1]], pbufs[slot], sems[slot])
        c3 = pltpu.async_copy(ie_h.at[idx_v.at[slot * 3 + 2]], nbufs[slot], sems[slot])
        return (c1, c2, c3)

    lane15 = lax.iota(jnp.int32, L) == (L - 1)

    pending = fetch(0, 0)
    for j in range(NCH):
        slot = j % 2
        current = pending
        if j + 1 < NCH:
            pending = fetch(j + 1, (j + 1) % 2)
        for c in current:
            c.wait()
        ub, pb, nb = ubufs[slot], pbufs[slot], nbufs[slot]

        def row(r, carry, ub=ub, pb=pb, nb=nb, j=j):
            accp = jnp.zeros((L,), jnp.float32)
            accn = jnp.zeros((L,), jnp.float32)
            for cc in range(D // L):
                uv = ub[r, pl.ds(cc * L, L)]
                pv = pb[r, pl.ds(cc * L, L)]
                nv = nb[r, pl.ds(cc * L, L)]
                accp = accp + uv * pv
                accn = accn + uv * nv
            out_idx = jnp.broadcast_to(j * CH + r, (L,)).astype(jnp.int32)
            plsc.store_scatter(pout, [out_idx], plsc.cumsum(accp), mask=lane15)
            plsc.store_scatter(nout, [out_idx], plsc.cumsum(accn), mask=lane15)
            return carry

        lax.fori_loop(0, CH, row, 0)

    pltpu.sync_copy(pout, pos_o.at[pl.ds(base, BPW)])
    pltpu.sync_copy(nout, neg_o.at[pl.ds(base, BPW)])


_bpr = pl.kernel(
    _bpr_body,
    out_type=[
        jax.ShapeDtypeStruct((B,), jnp.float32),
        jax.ShapeDtypeStruct((B,), jnp.float32),
    ],
    mesh=_MESH,
    compiler_params=pltpu.CompilerParams(needs_layout_passes=False),
    scratch_types=[
        pltpu.VMEM((6, CH), jnp.int32),
        pltpu.VMEM((CH, D), jnp.float32),
        pltpu.VMEM((CH, D), jnp.float32),
        pltpu.VMEM((CH, D), jnp.float32),
        pltpu.VMEM((CH, D), jnp.float32),
        pltpu.VMEM((CH, D), jnp.float32),
        pltpu.VMEM((CH, D), jnp.float32),
        pltpu.VMEM((BPW,), jnp.float32),
        pltpu.VMEM((BPW,), jnp.float32),
        pltpu.SemaphoreType.DMA,
        pltpu.SemaphoreType.DMA,
    ],
)


@jax.jit
def kernel(user, pos_item, neg_item, user_embedding, item_embedding):
    user = user.astype(jnp.int32)
    pos_item = pos_item.astype(jnp.int32)
    neg_item = neg_item.astype(jnp.int32)
    pos_pred, neg_pred = _bpr(user, pos_item, neg_item,
                              user_embedding, item_embedding)
    return (pos_pred, neg_pred)
